# baseline (speedup 1.0000x reference)
import jax
import jax.numpy as jnp
from jax.experimental import pallas as pl

NUM_LAYERS = 4
N_GRAPHS = 128


def _bn(u, gamma, beta):
    mean = u.mean(axis=0)
    var = u.var(axis=0)
    return (u - mean) / jnp.sqrt(var + 1e-5) * gamma + beta


def _pool_mean(u, batch):
    s = jax.ops.segment_sum(u, batch, num_segments=N_GRAPHS)
    cnt = jax.ops.segment_sum(jnp.ones((u.shape[0], 1), jnp.float32), batch, num_segments=N_GRAPHS)
    return s / jnp.maximum(cnt, 1.0)


def _extractor(u, batch, p):
    g = _pool_mean(u, batch)
    out = g @ p["W1"] + p["b1"]
    out = out + jax.nn.relu(out @ p["W2"] + p["b2"])
    return out


def _smp_layer(u, edge_index, edge_attr, p):
    src = edge_index[0]
    dst = edge_index[1]
    e = edge_attr @ p["We"] + p["be"]
    m = (u @ p["Wm"] + p["bm"])[src] * e
    agg = jax.ops.segment_sum(m, dst, num_segments=u.shape[0])
    deg = jax.ops.segment_sum(jnp.ones((edge_index.shape[1], 1), jnp.float32), dst, num_segments=u.shape[0])
    agg = agg / jnp.maximum(deg, 1.0)
    out = u @ p["Wu"] + agg @ p["Wa"] + p["bu"]
    return jax.nn.relu(out)


def kernel(x, edge_index, edge_attr, batch, params):
    u = x
    out = _extractor(u, batch, params["no_prop"])
    u = u @ params["init_W"] + params["init_b"]
    for i in range(NUM_LAYERS):
        if i > 0:
            u = _bn(u, params["bn_gamma"][i], params["bn_beta"][i])
        layer = {kk: params[kk][i] for kk in ["We", "be", "Wm", "bm", "Wu", "Wa", "bu"]}
        u = _smp_layer(u, edge_index, edge_attr, layer) + u
        out = out + _extractor(u, batch, params["fe"]) / NUM_LAYERS
    h = out @ params["after_W"] + params["after_b"]
    out = (jax.nn.relu(h) + out) @ params["final_W"] + params["final_b"]
    return out[:, 0]


# speedup vs baseline: 2.3787x; 2.3787x over previous
import functools

import jax
import jax.numpy as jnp
from jax import lax
from jax.experimental import pallas as pl
from jax.experimental.pallas import tpu as pltpu
from jax.experimental.pallas import tpu_sc as plsc

N = 10000
E = 320000
F_IN = 128
F_EDGE = 16
H = 64
L = 4
G = 128

NC = 2
NS = 16
TILES = NC * NS
EPT = E // TILES
CH = 80
NCHUNK = EPT // CH
NP_ = 10240
RPT = NP_ // NS
ZR = 128
VLANES = 16


def _head_body(x_ref, batch_ref, initW_ref, initb_ref, W1_ref, b1_ref,
               W2_ref, b2_ref, Wm0_ref, bm0_ref,
               out0_ref, u0_ref, um0_ref, pnt_ref):
    iota_g = lax.broadcasted_iota(jnp.int32, (G, N), 0)
    oh = (batch_ref[...] == iota_g).astype(jnp.float32)
    cnt = jnp.sum(oh, axis=1, keepdims=True)
    pnt = oh / jnp.maximum(cnt, 1.0)
    pnt_ref[...] = pnt
    x = x_ref[...]
    g0 = jnp.dot(pnt, x, preferred_element_type=jnp.float32)
    o = jnp.dot(g0, W1_ref[...], preferred_element_type=jnp.float32) + b1_ref[...]
    o = o + jax.nn.relu(
        jnp.dot(o, W2_ref[...], preferred_element_type=jnp.float32) + b2_ref[...])
    out0_ref[...] = o
    u0 = jnp.dot(x, initW_ref[...], preferred_element_type=jnp.float32) + initb_ref[...]
    u0_ref[...] = u0
    um0 = jnp.dot(u0, Wm0_ref[...], preferred_element_type=jnp.float32) + bm0_ref[...]
    um0_ref[...] = jnp.concatenate([um0, um0], axis=1)


def _head_call(x, batch2, initW, initb, W1, b1, W2, b2, Wm0, bm0):
    return pl.pallas_call(
        _head_body,
        out_shape=[
            jax.ShapeDtypeStruct((G, H), jnp.float32),
            jax.ShapeDtypeStruct((N, H), jnp.float32),
            jax.ShapeDtypeStruct((N, 2 * H), jnp.float32),
            jax.ShapeDtypeStruct((G, N), jnp.float32),
        ],
    )(x, batch2, initW, initb, W1, b1, W2, b2, Wm0, bm0)


EB2 = 4000


def _edge_body(ea2_ref, W2_ref, be2_ref, e2_ref):
    e2_ref[...] = jnp.dot(ea2_ref[...], W2_ref[...],
                          preferred_element_type=jnp.float32) + be2_ref[...]


def _edge_call(ea2, W2, be2):
    return pl.pallas_call(
        _edge_body,
        grid=(E // 2 // EB2,),
        in_specs=[
            pl.BlockSpec((EB2, 2 * F_EDGE), lambda i: (i, 0)),
            pl.BlockSpec((2 * F_EDGE, 2 * H), lambda i: (0, 0)),
            pl.BlockSpec((1, 2 * H), lambda i: (0, 0)),
        ],
        out_specs=pl.BlockSpec((EB2, 2 * H), lambda i: (i, 0)),
        out_shape=jax.ShapeDtypeStruct((E // 2, 2 * H), jnp.float32),
    )(ea2, W2, be2)


def _sc_body(um2, e2, srcs, dsts, agg_out,
             srcbuf, dstbuf, gbuf, ebuf, mbuf, zbuf, agg_sh, gsem):
    c = lax.axis_index("c")
    s = lax.axis_index("s")
    wid = c * NS + s
    ebase = wid * EPT

    zero16 = jnp.zeros((VLANES,), jnp.float32)
    one16 = jnp.ones((VLANES,), jnp.float32)

    def zrow(i, _):
        for j in range(2 * H // VLANES):
            zbuf[i, pl.ds(VLANES * j, VLANES)] = zero16
        return 0

    lax.fori_loop(0, ZR, zrow, 0)

    def orow(i, _):
        for j in range(H // VLANES):
            mbuf[i, pl.ds(H + VLANES * j, VLANES)] = one16
        return 0

    lax.fori_loop(0, CH, orow, 0)

    for k in range(RPT // ZR):
        row0 = s * RPT + k * ZR
        pltpu.sync_copy(zbuf, agg_sh.at[pl.ds(row0, ZR), :])

    plsc.subcore_barrier()

    def chunk(ci, _):
        base = pl.multiple_of(ebase + ci * CH, 8)
        base2 = pl.multiple_of((ebase + ci * CH) // 2, 8)
        pltpu.sync_copy(srcs.at[pl.ds(base, CH)], srcbuf)
        pltpu.sync_copy(dsts.at[pl.ds(base, CH)], dstbuf)
        pltpu.async_copy(um2.at[srcbuf], gbuf, gsem).wait()
        pltpu.sync_copy(e2.at[pl.ds(base2, CH // 2), :], ebuf)

        def mrow(i2, _):
            for jj in range(2 * H // VLANES):
                edge = 2 * i2 + (1 if jj >= H // VLANES else 0)
                fsl = pl.ds(VLANES * (jj % (H // VLANES)), VLANES)
                mbuf[edge, fsl] = ebuf[i2, pl.ds(VLANES * jj, VLANES)] * gbuf[edge, fsl]
            return 0

        lax.fori_loop(0, CH // 2, mrow, 0)
        pltpu.sync_copy(mbuf, agg_sh.at[dstbuf], add=True)
        return 0

    lax.fori_loop(0, NCHUNK, chunk, 0)
    plsc.subcore_barrier()

    row0 = pl.multiple_of(s * RPT, 8)
    pltpu.sync_copy(agg_sh.at[pl.ds(row0, RPT), :],
                    agg_out.at[c, pl.ds(row0, RPT), :])


def _sc_call(um2, e2, srcs, dsts):
    mesh = plsc.VectorSubcoreMesh(core_axis_name="c", subcore_axis_name="s")
    fn = pl.kernel(
        _sc_body,
        mesh=mesh,
        out_type=jax.ShapeDtypeStruct((NC, NP_, 2 * H), jnp.float32),
        scratch_types=[
            pltpu.VMEM((CH,), jnp.int32),
            pltpu.VMEM((CH,), jnp.int32),
            pltpu.VMEM((CH, 2 * H), jnp.float32),
            pltpu.VMEM((CH // 2, 2 * H), jnp.float32),
            pltpu.VMEM((CH, 2 * H), jnp.float32),
            pltpu.VMEM((ZR, 2 * H), jnp.float32),
            pltpu.VMEM_SHARED((NP_, 2 * H), jnp.float32),
            pltpu.SemaphoreType.DMA,
        ],
    )
    return fn(um2, e2, srcs, dsts)


def _combine_body(last, *refs):
    if last:
        (agg2_ref, u_ref, pnt_ref, outp_ref,
         Wu_ref, Wa_ref, bu_ref, W1_ref, b1_ref, W2_ref, b2_ref,
         aW_ref, ab_ref, fW_ref, fb_ref,
         res_ref) = refs
    else:
        (agg2_ref, u_ref, pnt_ref, outp_ref,
         Wu_ref, Wa_ref, bu_ref, W1_ref, b1_ref, W2_ref, b2_ref,
         g_ref, bt_ref, Wmn_ref, bmn_ref,
         ub_ref, umn_ref, outn_ref) = refs

    agg_full = agg2_ref[0] + agg2_ref[1]
    deg = agg_full[:N, H:H + 1]
    rdeg = 1.0 / jnp.maximum(deg, 1.0)
    agg = agg_full[:N, :H] * rdeg
    u = u_ref[...]
    unew = jax.nn.relu(
        jnp.dot(u, Wu_ref[...], preferred_element_type=jnp.float32)
        + jnp.dot(agg, Wa_ref[...], preferred_element_type=jnp.float32)
        + bu_ref[...]) + u
    gpool = jnp.dot(pnt_ref[...], unew, preferred_element_type=jnp.float32)
    o = jnp.dot(gpool, W1_ref[...], preferred_element_type=jnp.float32) + b1_ref[...]
    o = o + jax.nn.relu(
        jnp.dot(o, W2_ref[...], preferred_element_type=jnp.float32) + b2_ref[...])
    outn = outp_ref[...] + o * (1.0 / L)

    if last:
        h = jnp.dot(outn, aW_ref[...], preferred_element_type=jnp.float32) + ab_ref[...]
        res = jnp.dot(jax.nn.relu(h) + outn, fW_ref[...],
                      preferred_element_type=jnp.float32) + fb_ref[...]
        res_ref[...] = res
    else:
        mean = jnp.mean(unew, axis=0, keepdims=True)
        var = jnp.mean((unew - mean) ** 2, axis=0, keepdims=True)
        ub = (unew - mean) / jnp.sqrt(var + 1e-5) * g_ref[...] + bt_ref[...]
        ub_ref[...] = ub
        umn = jnp.dot(ub, Wmn_ref[...], preferred_element_type=jnp.float32) + bmn_ref[...]
        umn_ref[...] = jnp.concatenate([umn, umn], axis=1)
        outn_ref[...] = outn


def _combine_call(i, agg2, u, pnt, outp, p):
    last = i == L - 1
    fe = p["fe"]
    args = [agg2, u, pnt, outp,
            p["Wu"][i], p["Wa"][i], p["bu"][i].reshape(1, H),
            fe["W1"], fe["b1"].reshape(1, H), fe["W2"], fe["b2"].reshape(1, H)]
    if last:
        args += [p["after_W"], p["after_b"].reshape(1, H),
                 p["final_W"], p["final_b"].reshape(1, 1)]
        out_shape = [jax.ShapeDtypeStruct((G, 1), jnp.float32)]
    else:
        args += [p["bn_gamma"][i + 1].reshape(1, H), p["bn_beta"][i + 1].reshape(1, H),
                 p["Wm"][i + 1], p["bm"][i + 1].reshape(1, H)]
        out_shape = [
            jax.ShapeDtypeStruct((N, H), jnp.float32),
            jax.ShapeDtypeStruct((N, 2 * H), jnp.float32),
            jax.ShapeDtypeStruct((G, H), jnp.float32),
        ]
    return pl.pallas_call(
        functools.partial(_combine_body, last),
        out_shape=out_shape,
    )(*args)


def kernel(x, edge_index, edge_attr, batch, params):
    p = params
    src = edge_index[0]
    dst = edge_index[1]
    batch2 = batch.reshape(1, N)
    np_ = p["no_prop"]
    out0, u0, um0, pnt = _head_call(
        x, batch2, p["init_W"], p["init_b"].reshape(1, H),
        np_["W1"], np_["b1"].reshape(1, H), np_["W2"], np_["b2"].reshape(1, H),
        p["Wm"][0], p["bm"][0].reshape(1, H))

    ea2 = edge_attr.reshape(E // 2, 2 * F_EDGE)
    u, um, out = u0, um0, out0
    for i in range(L):
        We = p["We"][i]
        W2 = jnp.zeros((2 * F_EDGE, 2 * H), jnp.float32)
        W2 = W2.at[:F_EDGE, :H].set(We).at[F_EDGE:, H:].set(We)
        be2 = jnp.concatenate([p["be"][i], p["be"][i]]).reshape(1, 2 * H)
        e2 = _edge_call(ea2, W2, be2)
        agg2 = _sc_call(um, e2, src, dst)
        if i < L - 1:
            u, um, out = _combine_call(i, agg2, u, pnt, out, p)
        else:
            (res,) = _combine_call(i, agg2, u, pnt, out, p)
    return res[:, 0]


# speedup vs baseline: 5.0839x; 2.1372x over previous
import functools

import jax
import jax.numpy as jnp
from jax import lax
from jax.experimental import pallas as pl
from jax.experimental.pallas import tpu as pltpu
from jax.experimental.pallas import tpu_sc as plsc

N = 10000
E = 320000
F_IN = 128
F_EDGE = 16
H = 64
L = 4
G = 128

NC = 2
NS = 16
TILES = NC * NS
EPT = E // TILES
CH = 80
NCHUNK = EPT // CH
NP_ = 10240
RPT = NP_ // NS
ZR = 88
VLANES = 16
AW = 80
NPC = 5120
NPD = 5632
RPTA = NPD // NS
RPTO = NPC // NS


def _head_body(x_ref, batch_ref, initW_ref, initb_ref, W1_ref, b1_ref,
               W2_ref, b2_ref, Wm0_ref, bm0_ref,
               out0_ref, u0_ref, um0_ref, pnt_ref):
    iota_g = lax.broadcasted_iota(jnp.int32, (G, N), 0)
    oh = (batch_ref[...] == iota_g).astype(jnp.float32)
    cnt = jnp.sum(oh, axis=1, keepdims=True)
    pnt = oh / jnp.maximum(cnt, 1.0)
    pnt_ref[...] = pnt
    x = x_ref[...]
    g0 = jnp.dot(pnt, x, preferred_element_type=jnp.float32)
    o = jnp.dot(g0, W1_ref[...], preferred_element_type=jnp.float32) + b1_ref[...]
    o = o + jax.nn.relu(
        jnp.dot(o, W2_ref[...], preferred_element_type=jnp.float32) + b2_ref[...])
    out0_ref[...] = o
    u0 = jnp.dot(x, initW_ref[...], preferred_element_type=jnp.float32) + initb_ref[...]
    u0_ref[...] = u0
    um0 = jnp.dot(u0, Wm0_ref[...], preferred_element_type=jnp.float32) + bm0_ref[...]
    um0_ref[...] = jnp.concatenate([um0, um0], axis=1)


def _head_call(x, batch2, initW, initb, W1, b1, W2, b2, Wm0, bm0):
    return pl.pallas_call(
        _head_body,
        out_shape=[
            jax.ShapeDtypeStruct((G, H), jnp.float32),
            jax.ShapeDtypeStruct((N, H), jnp.float32),
            jax.ShapeDtypeStruct((N, 2 * H), jnp.float32),
            jax.ShapeDtypeStruct((G, N), jnp.float32),
        ],
    )(x, batch2, initW, initb, W1, b1, W2, b2, Wm0, bm0)


EB2 = 4000


def _edge_body(ea2_ref, W2_ref, be2_ref, e2_ref):
    e2_ref[...] = jnp.dot(ea2_ref[...], W2_ref[...],
                          preferred_element_type=jnp.float32) + be2_ref[...]


def _edge_call(ea2, W2, be2):
    return pl.pallas_call(
        _edge_body,
        grid=(E // 2 // EB2,),
        in_specs=[
            pl.BlockSpec((EB2, 2 * F_EDGE), lambda i: (i, 0)),
            pl.BlockSpec((2 * F_EDGE, 2 * H), lambda i: (0, 0)),
            pl.BlockSpec((1, 2 * H), lambda i: (0, 0)),
        ],
        out_specs=pl.BlockSpec((EB2, 2 * H), lambda i: (i, 0)),
        out_shape=jax.ShapeDtypeStruct((E // 2, 2 * H), jnp.float32),
    )(ea2, W2, be2)


def _sc_body(um2, e2, src3, dst3, agg_out,
             srcall, dstall, gbuf0, gbuf1, ebuf0, ebuf1, mbuf, zbuf, agg_sh,
             gsem0, gsem1, esem0, esem1):
    c = lax.axis_index("c")
    s = lax.axis_index("s")
    wid = c * NS + s
    ebase2 = wid * (EPT // 2)
    lo = c * NPC

    zero16 = jnp.zeros((VLANES,), jnp.float32)
    one16 = jnp.ones((VLANES,), jnp.float32)
    dummy16 = NPC + lax.iota(jnp.int32, VLANES) * 8

    pltpu.sync_copy(src3.at[wid], srcall)
    pltpu.sync_copy(dst3.at[wid], dstall)

    def drow(i, _):
        for j in range(CH // VLANES):
            sl = pl.ds(VLANES * j, VLANES)
            lv = dstall[i, sl] - lo
            ok = (lv >= 0) & (lv < NPC)
            dstall[i, sl] = jnp.where(ok, lv, dummy16)
        return 0

    lax.fori_loop(0, NCHUNK, drow, 0)

    def zrow(i, _):
        for j in range(AW // VLANES):
            zbuf[i, pl.ds(VLANES * j, VLANES)] = zero16
        return 0

    lax.fori_loop(0, ZR, zrow, 0)

    def orow(i, _):
        mbuf[i, pl.ds(H, VLANES)] = one16
        return 0

    lax.fori_loop(0, CH, orow, 0)

    for k in range(RPTA // ZR):
        row0 = s * RPTA + k * ZR
        pltpu.sync_copy(zbuf, agg_sh.at[pl.ds(row0, ZR), :])

    plsc.subcore_barrier()

    def issue(ci, gb, eb, gsem, esem):
        pltpu.async_copy(um2.at[srcall.at[ci]], gb, gsem)
        base2 = pl.multiple_of(ebase2 + ci * (CH // 2), 8)
        pltpu.async_copy(e2.at[pl.ds(base2, CH // 2), :], eb, esem)

    def wait_ge(gb, eb, gsem, esem):
        pltpu.make_async_copy(um2.at[pl.ds(0, CH), :], gb, gsem).wait()
        pltpu.make_async_copy(e2.at[pl.ds(0, CH // 2), :], eb, esem).wait()

    def compute_scatter(ci, gb, eb):
        def mrow(k2, _):
            for r in range(4):
                i2 = 4 * k2 + r
                for jj in range(2 * H // VLANES):
                    edge = 2 * i2 + (1 if jj >= H // VLANES else 0)
                    fsl = pl.ds(VLANES * (jj % (H // VLANES)), VLANES)
                    mbuf[edge, fsl] = (eb[i2, pl.ds(VLANES * jj, VLANES)]
                                       * gb[edge, fsl])
            return 0

        lax.fori_loop(0, CH // 8, mrow, 0)
        pltpu.sync_copy(mbuf, agg_sh.at[dstall.at[ci]], add=True)

    def step(ci, cur, nxt, issue_next):
        (gb, eb, gsem, esem) = cur
        wait_ge(gb, eb, gsem, esem)
        if issue_next:
            issue(ci + 1, *nxt)
        compute_scatter(ci, gb, eb)

    B0 = (gbuf0, ebuf0, gsem0, esem0)
    B1 = (gbuf1, ebuf1, gsem1, esem1)

    issue(0, *B0)

    def pair(k, _):
        step(2 * k, B0, B1, True)
        step(2 * k + 1, B1, B0, True)
        return 0

    lax.fori_loop(0, (NCHUNK - 1) // 2, pair, 0)
    step(NCHUNK - 1, B0, B1, False)

    plsc.subcore_barrier()

    row0 = pl.multiple_of(s * RPTO, 8)
    orow0 = pl.multiple_of(c * NPC + s * RPTO, 8)
    pltpu.sync_copy(agg_sh.at[pl.ds(row0, RPTO), :],
                    agg_out.at[pl.ds(orow0, RPTO), :])


def _sc_call(um2, e2, src3, dst3):
    mesh = plsc.VectorSubcoreMesh(core_axis_name="c", subcore_axis_name="s")
    fn = pl.kernel(
        _sc_body,
        mesh=mesh,
        out_type=jax.ShapeDtypeStruct((NP_, AW), jnp.float32),
        scratch_types=[
            pltpu.VMEM((NCHUNK, CH), jnp.int32),
            pltpu.VMEM((NCHUNK, CH), jnp.int32),
            pltpu.VMEM((CH, 2 * H), jnp.float32),
            pltpu.VMEM((CH, 2 * H), jnp.float32),
            pltpu.VMEM((CH // 2, 2 * H), jnp.float32),
            pltpu.VMEM((CH // 2, 2 * H), jnp.float32),
            pltpu.VMEM((CH, AW), jnp.float32),
            pltpu.VMEM((ZR, AW), jnp.float32),
            pltpu.VMEM_SHARED((NPD, AW), jnp.float32),
            pltpu.SemaphoreType.DMA,
            pltpu.SemaphoreType.DMA,
            pltpu.SemaphoreType.DMA,
            pltpu.SemaphoreType.DMA,
        ],
    )
    return fn(um2, e2, src3, dst3)


def _combine_body(last, *refs):
    if last:
        (agg2_ref, u_ref, pnt_ref, outp_ref,
         Wu_ref, Wa_ref, bu_ref, W1_ref, b1_ref, W2_ref, b2_ref,
         aW_ref, ab_ref, fW_ref, fb_ref,
         res_ref) = refs
    else:
        (agg2_ref, u_ref, pnt_ref, outp_ref,
         Wu_ref, Wa_ref, bu_ref, W1_ref, b1_ref, W2_ref, b2_ref,
         g_ref, bt_ref, Wmn_ref, bmn_ref,
         ub_ref, umn_ref, outn_ref) = refs

    agg_full = agg2_ref[:N]
    deg = agg_full[:, H:H + 1]
    rdeg = 1.0 / jnp.maximum(deg, 1.0)
    agg = agg_full[:, :H] * rdeg
    u = u_ref[...]
    unew = jax.nn.relu(
        jnp.dot(u, Wu_ref[...], preferred_element_type=jnp.float32)
        + jnp.dot(agg, Wa_ref[...], preferred_element_type=jnp.float32)
        + bu_ref[...]) + u
    gpool = jnp.dot(pnt_ref[...], unew, preferred_element_type=jnp.float32)
    o = jnp.dot(gpool, W1_ref[...], preferred_element_type=jnp.float32) + b1_ref[...]
    o = o + jax.nn.relu(
        jnp.dot(o, W2_ref[...], preferred_element_type=jnp.float32) + b2_ref[...])
    outn = outp_ref[...] + o * (1.0 / L)

    if last:
        h = jnp.dot(outn, aW_ref[...], preferred_element_type=jnp.float32) + ab_ref[...]
        res = jnp.dot(jax.nn.relu(h) + outn, fW_ref[...],
                      preferred_element_type=jnp.float32) + fb_ref[...]
        res_ref[...] = res
    else:
        mean = jnp.mean(unew, axis=0, keepdims=True)
        var = jnp.mean((unew - mean) ** 2, axis=0, keepdims=True)
        ub = (unew - mean) / jnp.sqrt(var + 1e-5) * g_ref[...] + bt_ref[...]
        ub_ref[...] = ub
        umn = jnp.dot(ub, Wmn_ref[...], preferred_element_type=jnp.float32) + bmn_ref[...]
        umn_ref[...] = jnp.concatenate([umn, umn], axis=1)
        outn_ref[...] = outn


def _combine_call(i, agg2, u, pnt, outp, p):
    last = i == L - 1
    fe = p["fe"]
    args = [agg2, u, pnt, outp,
            p["Wu"][i], p["Wa"][i], p["bu"][i].reshape(1, H),
            fe["W1"], fe["b1"].reshape(1, H), fe["W2"], fe["b2"].reshape(1, H)]
    if last:
        args += [p["after_W"], p["after_b"].reshape(1, H),
                 p["final_W"], p["final_b"].reshape(1, 1)]
        out_shape = [jax.ShapeDtypeStruct((G, 1), jnp.float32)]
    else:
        args += [p["bn_gamma"][i + 1].reshape(1, H), p["bn_beta"][i + 1].reshape(1, H),
                 p["Wm"][i + 1], p["bm"][i + 1].reshape(1, H)]
        out_shape = [
            jax.ShapeDtypeStruct((N, H), jnp.float32),
            jax.ShapeDtypeStruct((N, 2 * H), jnp.float32),
            jax.ShapeDtypeStruct((G, H), jnp.float32),
        ]
    return pl.pallas_call(
        functools.partial(_combine_body, last),
        out_shape=out_shape,
    )(*args)


def kernel(x, edge_index, edge_attr, batch, params):
    p = params
    src3 = edge_index[0].reshape(TILES, NCHUNK, CH)
    dst3 = edge_index[1].reshape(TILES, NCHUNK, CH)
    batch2 = batch.reshape(1, N)
    np_ = p["no_prop"]
    out0, u0, um0, pnt = _head_call(
        x, batch2, p["init_W"], p["init_b"].reshape(1, H),
        np_["W1"], np_["b1"].reshape(1, H), np_["W2"], np_["b2"].reshape(1, H),
        p["Wm"][0], p["bm"][0].reshape(1, H))

    ea2 = edge_attr.reshape(E // 2, 2 * F_EDGE)
    u, um, out = u0, um0, out0
    for i in range(L):
        We = p["We"][i]
        W2 = jnp.zeros((2 * F_EDGE, 2 * H), jnp.float32)
        W2 = W2.at[:F_EDGE, :H].set(We).at[F_EDGE:, H:].set(We)
        be2 = jnp.concatenate([p["be"][i], p["be"][i]]).reshape(1, 2 * H)
        e2 = _edge_call(ea2, W2, be2)
        agg2 = _sc_call(um, e2, src3, dst3)
        if i < L - 1:
            u, um, out = _combine_call(i, agg2, u, pnt, out, p)
        else:
            (res,) = _combine_call(i, agg2, u, pnt, out, p)
    return res[:, 0]
